# baseline (speedup 1.0000x reference)
import functools

import jax
import jax.numpy as jnp
from jax import lax
from jax.experimental import pallas as pl
from jax.experimental.pallas import tpu as pltpu
from jax.experimental.pallas import tpu_sc as plsc

N_ROWS = 1000
EMBED_DIM = 32
LANES = 16
NUM_CORES = 2
NUM_SUBCORES = 16
NUM_WORKERS = NUM_CORES * NUM_SUBCORES

B, T, H, W = 8, 20, 64, 64
T_PER_W = (B * T) // NUM_WORKERS
STRIPE = 8
N_STRIPES = H // STRIPE
N_CHUNKS = T_PER_W * N_STRIPES
W_GRPS = W // LANES


def _sc_gather(x, table):
  mesh = plsc.VectorSubcoreMesh(
      core_axis_name="c", subcore_axis_name="s",
      num_cores=NUM_CORES, num_subcores=NUM_SUBCORES)

  @functools.partial(
      pl.kernel,
      mesh=mesh,
      out_type=jax.ShapeDtypeStruct((B, T, H, EMBED_DIM, W), jnp.float32),
      scratch_types=[
          pltpu.VMEM_SHARED((N_ROWS, EMBED_DIM), jnp.float32),
          pltpu.VMEM((2, STRIPE, W), jnp.int32),
          pltpu.VMEM((2, STRIPE, W, EMBED_DIM), jnp.float32),
          pltpu.VMEM((2, STRIPE, EMBED_DIM, W), jnp.float32),
          pltpu.SemaphoreType.DMA((2,)),
          pltpu.SemaphoreType.DMA((2,)),
          pltpu.SemaphoreType.DMA((2,)),
      ],
      compiler_params=pltpu.CompilerParams(
          use_tc_tiling_on_sc=False, needs_layout_passes=False),
  )
  def k(x_hbm, table_hbm, out_hbm, table_sh, idx_v, rows_v, t_v,
        isem, gsem, osem):
    wid = lax.axis_index("s") * NUM_CORES + lax.axis_index("c")
    b = wid // (NUM_WORKERS // B)
    t0 = (wid % (NUM_WORKERS // B)) * T_PER_W

    @pl.when(lax.axis_index("s") == 0)
    def _():
      pltpu.sync_copy(table_hbm, table_sh)

    def idx_copy(i, s):
      p = i // N_STRIPES
      q = i % N_STRIPES
      return pltpu.make_async_copy(
          x_hbm.at[b, t0 + p, pl.ds(q * STRIPE, STRIPE)],
          idx_v.at[s], isem.at[s])

    def gather_copies(s):
      return [
          pltpu.make_async_copy(
              table_sh.at[idx_v.at[s, r]], rows_v.at[s, r], gsem.at[s])
          for r in range(STRIPE)
      ]

    def out_copy(i, s):
      p = i // N_STRIPES
      q = i % N_STRIPES
      return pltpu.make_async_copy(
          t_v.at[s],
          out_hbm.at[b, t0 + p, pl.ds(q * STRIPE, STRIPE)],
          osem.at[s])

    lane = lax.iota(jnp.int32, LANES)
    zero16 = jnp.zeros((LANES,), jnp.int32)

    def transpose(s):
      def body(j, carry):
        r = j // W_GRPS
        w0 = (j % W_GRPS) * LANES
        idx_r = zero16 + r
        idx_w = w0 + lane
        for e in range(EMBED_DIM):
          v = plsc.load_gather(
              rows_v.at[s], [idx_r, idx_w, zero16 + e])
          t_v[s, r, e, pl.ds(w0, LANES)] = v
        return carry

      lax.fori_loop(0, STRIPE * W_GRPS, body, 0)

    idx_copy(0, 0).start()
    idx_copy(1, 1).start()
    plsc.subcore_barrier()
    idx_copy(0, 0).wait()
    for g in gather_copies(0):
      g.start()

    def step(kk, carry):
      a = 2 * kk
      bb = a + 1
      not_last = kk < N_CHUNKS // 2 - 1

      for g in gather_copies(0):
        g.wait()

      @pl.when(not_last)
      def _():
        idx_copy(a + 2, 0).start()

      idx_copy(bb, 1).wait()
      for g in gather_copies(1):
        g.start()

      @pl.when(kk > 0)
      def _():
        out_copy(a - 2, 0).wait()

      transpose(0)
      out_copy(a, 0).start()

      @pl.when(not_last)
      def _():
        idx_copy(a + 2, 0).wait()
        for g in gather_copies(0):
          g.start()

      for g in gather_copies(1):
        g.wait()

      @pl.when(not_last)
      def _():
        idx_copy(bb + 2, 1).start()

      @pl.when(kk > 0)
      def _():
        out_copy(bb - 2, 1).wait()

      transpose(1)
      out_copy(bb, 1).start()

      return carry

    lax.fori_loop(0, N_CHUNKS // 2, step, 0)

    out_copy(N_CHUNKS - 2, 0).wait()
    out_copy(N_CHUNKS - 1, 1).wait()

  return k(x, table)


def kernel(x, table):
  out = _sc_gather(x, table)
  return jnp.transpose(out, (0, 1, 2, 4, 3))


# speedup vs baseline: 2.1461x; 2.1461x over previous
import functools

import jax
import jax.numpy as jnp
from jax import lax
from jax.experimental import pallas as pl
from jax.experimental.pallas import tpu as pltpu
from jax.experimental.pallas import tpu_sc as plsc

N_ROWS = 1000
EMBED_DIM = 32
LANES = 16
NUM_CORES = 2
NUM_SUBCORES = 16
NUM_WORKERS = NUM_CORES * NUM_SUBCORES

B, T, H, W = 8, 20, 64, 64
T_PER_W = (B * T) // NUM_WORKERS
STRIPE = 8
N_STRIPES = H // STRIPE
N_CHUNKS = T_PER_W * N_STRIPES
WP = W + 1


def _sc_gather(x, table):
  mesh = plsc.VectorSubcoreMesh(
      core_axis_name="c", subcore_axis_name="s",
      num_cores=NUM_CORES, num_subcores=NUM_SUBCORES)

  @functools.partial(
      pl.kernel,
      mesh=mesh,
      out_type=jax.ShapeDtypeStruct((B, T, H, EMBED_DIM, W), jnp.float32),
      scratch_types=[
          pltpu.VMEM_SHARED((N_ROWS, EMBED_DIM), jnp.float32),
          pltpu.VMEM((2, STRIPE, W), jnp.int32),
          pltpu.VMEM((2, STRIPE, W, EMBED_DIM), jnp.float32),
          pltpu.VMEM((2, STRIPE, EMBED_DIM, WP), jnp.float32),
          pltpu.SemaphoreType.DMA((2,)),
          pltpu.SemaphoreType.DMA((2,)),
          pltpu.SemaphoreType.DMA((2,)),
      ],
      compiler_params=pltpu.CompilerParams(
          use_tc_tiling_on_sc=False, needs_layout_passes=False),
  )
  def k(x_hbm, table_hbm, out_hbm, table_sh, idx_v, rows_v, t_v,
        isem, gsem, osem):
    wid = lax.axis_index("s") * NUM_CORES + lax.axis_index("c")
    b = wid // (NUM_WORKERS // B)
    t0 = (wid % (NUM_WORKERS // B)) * T_PER_W

    @pl.when(lax.axis_index("s") == 0)
    def _():
      pltpu.sync_copy(table_hbm, table_sh)

    def idx_copy(i, s):
      p = i // N_STRIPES
      q = i % N_STRIPES
      return pltpu.make_async_copy(
          x_hbm.at[b, t0 + p, pl.ds(q * STRIPE, STRIPE)],
          idx_v.at[s], isem.at[s])

    def gather_copies(s):
      return [
          pltpu.make_async_copy(
              table_sh.at[idx_v.at[s, r]], rows_v.at[s, r], gsem.at[s])
          for r in range(STRIPE)
      ]

    def out_copy(i, s):
      p = i // N_STRIPES
      q = i % N_STRIPES
      return pltpu.make_async_copy(
          t_v.at[s, :, :, pl.ds(0, W)],
          out_hbm.at[b, t0 + p, pl.ds(q * STRIPE, STRIPE)],
          osem.at[s])

    lane = lax.iota(jnp.int32, LANES)
    zero16 = jnp.zeros((LANES,), jnp.int32)
    e_lo = lane
    e_hi = lane + LANES

    def transpose(s):
      def body(j, carry):
        r = j // W
        w = j % W
        idx_w = zero16 + w
        v_lo = rows_v[s, r, w, pl.ds(0, LANES)]
        v_hi = rows_v[s, r, w, pl.ds(LANES, LANES)]
        plsc.store_scatter(t_v.at[s, r], [e_lo, idx_w], v_lo)
        plsc.store_scatter(t_v.at[s, r], [e_hi, idx_w], v_hi)
        return carry

      lax.fori_loop(0, STRIPE * W, body, 0)

    idx_copy(0, 0).start()
    idx_copy(1, 1).start()
    plsc.subcore_barrier()
    idx_copy(0, 0).wait()
    for g in gather_copies(0):
      g.start()

    def step(kk, carry):
      a = 2 * kk
      bb = a + 1
      not_last = kk < N_CHUNKS // 2 - 1

      for g in gather_copies(0):
        g.wait()

      @pl.when(not_last)
      def _():
        idx_copy(a + 2, 0).start()

      idx_copy(bb, 1).wait()
      for g in gather_copies(1):
        g.start()

      @pl.when(kk > 0)
      def _():
        out_copy(a - 2, 0).wait()

      transpose(0)
      out_copy(a, 0).start()

      @pl.when(not_last)
      def _():
        idx_copy(a + 2, 0).wait()
        for g in gather_copies(0):
          g.start()

      for g in gather_copies(1):
        g.wait()

      @pl.when(not_last)
      def _():
        idx_copy(bb + 2, 1).start()

      @pl.when(kk > 0)
      def _():
        out_copy(bb - 2, 1).wait()

      transpose(1)
      out_copy(bb, 1).start()

      return carry

    lax.fori_loop(0, N_CHUNKS // 2, step, 0)

    out_copy(N_CHUNKS - 2, 0).wait()
    out_copy(N_CHUNKS - 1, 1).wait()

  return k(x, table)


def kernel(x, table):
  out = _sc_gather(x, table)
  return jnp.transpose(out, (0, 1, 2, 4, 3))


# speedup vs baseline: 2.2257x; 1.0371x over previous
import functools

import jax
import jax.numpy as jnp
from jax import lax
from jax.experimental import pallas as pl
from jax.experimental.pallas import tpu as pltpu
from jax.experimental.pallas import tpu_sc as plsc

N_ROWS = 1000
EMBED_DIM = 32
LANES = 16
NUM_CORES = 2
NUM_SUBCORES = 16
NUM_WORKERS = NUM_CORES * NUM_SUBCORES

B, T, H, W = 8, 20, 64, 64
T_PER_W = (B * T) // NUM_WORKERS
STRIPE = 8
N_STRIPES = H // STRIPE
N_CHUNKS = T_PER_W * N_STRIPES
WP = W + 1


def _sc_gather(x, table):
  mesh = plsc.VectorSubcoreMesh(
      core_axis_name="c", subcore_axis_name="s",
      num_cores=NUM_CORES, num_subcores=NUM_SUBCORES)

  @functools.partial(
      pl.kernel,
      mesh=mesh,
      out_type=jax.ShapeDtypeStruct((B, T, H, EMBED_DIM, W), jnp.float32),
      scratch_types=[
          pltpu.VMEM_SHARED((N_ROWS, EMBED_DIM), jnp.float32),
          pltpu.VMEM((2, STRIPE, W), jnp.int32),
          pltpu.VMEM((2, STRIPE, W, EMBED_DIM), jnp.float32),
          pltpu.VMEM((2, STRIPE, EMBED_DIM, WP), jnp.float32),
          pltpu.SemaphoreType.DMA((2,)),
          pltpu.SemaphoreType.DMA((2,)),
          pltpu.SemaphoreType.DMA((2,)),
      ],
      compiler_params=pltpu.CompilerParams(
          use_tc_tiling_on_sc=False, needs_layout_passes=False),
  )
  def k(x_hbm, table_hbm, out_hbm, table_sh, idx_v, rows_v, t_v,
        isem, gsem, osem):
    wid = lax.axis_index("s") * NUM_CORES + lax.axis_index("c")
    b = wid // (NUM_WORKERS // B)
    t0 = (wid % (NUM_WORKERS // B)) * T_PER_W

    @pl.when(lax.axis_index("s") == 0)
    def _():
      pltpu.sync_copy(table_hbm, table_sh)

    def idx_copy(i, s):
      p = i // N_STRIPES
      q = i % N_STRIPES
      return pltpu.make_async_copy(
          x_hbm.at[b, t0 + p, pl.ds(q * STRIPE, STRIPE)],
          idx_v.at[s], isem.at[s])

    def gather_copies(s):
      return [
          pltpu.make_async_copy(
              table_sh.at[idx_v.at[s, r]], rows_v.at[s, r], gsem.at[s])
          for r in range(STRIPE)
      ]

    def out_copy(i, s):
      p = i // N_STRIPES
      q = i % N_STRIPES
      return pltpu.make_async_copy(
          t_v.at[s, :, :, pl.ds(0, W)],
          out_hbm.at[b, t0 + p, pl.ds(q * STRIPE, STRIPE)],
          osem.at[s])

    lane = lax.iota(jnp.int32, LANES)
    zero16 = jnp.zeros((LANES,), jnp.int32)
    e_lo = lane
    e_hi = lane + LANES

    UNROLL = 8

    def transpose(s):
      def body(j, carry):
        r = j // (W // UNROLL)
        w0 = (j % (W // UNROLL)) * UNROLL
        for u in range(UNROLL):
          w = w0 + u
          idx_w = zero16 + w
          v_lo = rows_v[s, r, w, pl.ds(0, LANES)]
          v_hi = rows_v[s, r, w, pl.ds(LANES, LANES)]
          plsc.store_scatter(t_v.at[s, r], [e_lo, idx_w], v_lo)
          plsc.store_scatter(t_v.at[s, r], [e_hi, idx_w], v_hi)
        return carry

      lax.fori_loop(0, STRIPE * W // UNROLL, body, 0)

    idx_copy(0, 0).start()
    idx_copy(1, 1).start()
    plsc.subcore_barrier()
    idx_copy(0, 0).wait()
    for g in gather_copies(0):
      g.start()

    def step(kk, carry):
      a = 2 * kk
      bb = a + 1
      not_last = kk < N_CHUNKS // 2 - 1

      for g in gather_copies(0):
        g.wait()

      @pl.when(not_last)
      def _():
        idx_copy(a + 2, 0).start()

      idx_copy(bb, 1).wait()
      for g in gather_copies(1):
        g.start()

      @pl.when(kk > 0)
      def _():
        out_copy(a - 2, 0).wait()

      transpose(0)
      out_copy(a, 0).start()

      @pl.when(not_last)
      def _():
        idx_copy(a + 2, 0).wait()
        for g in gather_copies(0):
          g.start()

      for g in gather_copies(1):
        g.wait()

      @pl.when(not_last)
      def _():
        idx_copy(bb + 2, 1).start()

      @pl.when(kk > 0)
      def _():
        out_copy(bb - 2, 1).wait()

      transpose(1)
      out_copy(bb, 1).start()

      return carry

    lax.fori_loop(0, N_CHUNKS // 2, step, 0)

    out_copy(N_CHUNKS - 2, 0).wait()
    out_copy(N_CHUNKS - 1, 1).wait()

  return k(x, table)


def kernel(x, table):
  out = _sc_gather(x, table)
  return jnp.transpose(out, (0, 1, 2, 4, 3))
